# Initial kernel scaffold; baseline (speedup 1.0000x reference)
import jax
import jax.numpy as jnp
from jax.experimental import pallas as pl


def kernel(x, edge_weight, motif_val, W_conv, root, bias, wa, ba, motif_w, motif_b, edge_index, motif_idx):
    raise NotImplementedError("write your pallas kernel here")


# speedup vs baseline: 1.0015x; 1.0015x over previous
import functools
import jax
import jax.numpy as jnp
from jax import lax
from jax.experimental import pallas as pl
from jax.experimental.pallas import tpu as pltpu

N = 10000
E = 320000
C = 128
D = 32
NM = 13
NNZ = 320000
H = 64
RB = 1000
NRB = N // RB


def _dense_a_kernel(x_ref, w_ref, r_ref, b_ref, xw_ref, base_ref):
    xb = x_ref[...]
    xw_ref[0] = jnp.dot(xb, w_ref[0], preferred_element_type=jnp.float32)
    base_ref[0] = jnp.dot(xb, r_ref[0], preferred_element_type=jnp.float32) + b_ref[0]


def _dense_a(x, W_conv, root, bias):
    w_split = W_conv.reshape(C, 2, H).transpose(1, 0, 2)
    r_split = root.reshape(C, 2, H).transpose(1, 0, 2)
    b_split = bias.reshape(2, 1, H)
    return pl.pallas_call(
        _dense_a_kernel,
        grid=(2, NRB),
        in_specs=[
            pl.BlockSpec((RB, C), lambda c, i: (i, 0)),
            pl.BlockSpec((1, C, H), lambda c, i: (c, 0, 0)),
            pl.BlockSpec((1, C, H), lambda c, i: (c, 0, 0)),
            pl.BlockSpec((1, 1, H), lambda c, i: (c, 0, 0)),
        ],
        out_specs=[
            pl.BlockSpec((1, RB, H), lambda c, i: (c, i, 0)),
            pl.BlockSpec((1, RB, H), lambda c, i: (c, i, 0)),
        ],
        out_shape=[
            jax.ShapeDtypeStruct((2, N, H), jnp.float32),
            jax.ShapeDtypeStruct((2, N, H), jnp.float32),
        ],
    )(x, w_split, r_split, b_split)


def _attn_kernel(h_ref, s_ref, vc_ref, cb_ref, mb_ref, o_ref):
    acc = jnp.zeros((RB, 2 * NM * D), jnp.float32)
    for m in range(NM + 1):
        if m == 0:
            rm = jnp.concatenate([h_ref[0], h_ref[1]], axis=1)
        else:
            rm = jnp.concatenate([s_ref[m - 1, 0], s_ref[m - 1, 1]], axis=1)
        acc = acc + jnp.dot(rm, vc_ref[m], preferred_element_type=jnp.float32)
    zc = acc[:, : NM * D] + cb_ref[...]
    zm = acc[:, NM * D :] + mb_ref[...]
    g = (lax.broadcasted_iota(jnp.int32, (NM * D, NM), 0) // D
         == lax.broadcasted_iota(jnp.int32, (NM * D, NM), 1)).astype(jnp.float32)
    logits = jnp.dot(zc * zm, g, preferred_element_type=jnp.float32)
    att = jax.nn.sigmoid(logits)
    att_e = jnp.dot(att, g.T, preferred_element_type=jnp.float32)
    o_ref[...] = att_e * (zm - zc)


def _attn(h_pair, s_all, vc, cb, mb):
    return pl.pallas_call(
        _attn_kernel,
        grid=(NRB,),
        in_specs=[
            pl.BlockSpec((2, RB, H), lambda i: (0, i, 0)),
            pl.BlockSpec((NM, 2, RB, H), lambda i: (0, 0, i, 0)),
            pl.BlockSpec((NM + 1, C, 2 * NM * D), lambda i: (0, 0, 0)),
            pl.BlockSpec((1, NM * D), lambda i: (0, 0)),
            pl.BlockSpec((1, NM * D), lambda i: (0, 0)),
        ],
        out_specs=pl.BlockSpec((RB, NM * D), lambda i: (i, 0)),
        out_shape=jax.ShapeDtypeStruct((N, NM * D), jnp.float32),
    )(h_pair, s_all, vc, cb, mb)


def _build_combined_weights(wa, motif_w):
    blocks = motif_w.reshape(NM, NM, C, D)
    vc = jnp.zeros((NM + 1, C, 2 * NM * D), jnp.float32)
    for i in range(1, NM + 1):
        for j in range(NM + 1):
            if j == i:
                continue
            jj = j if j < i else j - 1
            vc = vc.at[j, :, (i - 1) * D : i * D].set(blocks[i - 1, jj])
        vc = vc.at[i, :, NM * D + (i - 1) * D : NM * D + i * D].set(wa)
    return vc


def kernel(x, edge_weight, motif_val, W_conv, root, bias, wa, ba, motif_w, motif_b, edge_index, motif_idx):
    xw_pair, base_pair = _dense_a(x, W_conv, root, bias)
    xw = jnp.concatenate([xw_pair[0], xw_pair[1]], axis=1)
    base = jnp.concatenate([base_pair[0], base_pair[1]], axis=1)

    src = edge_index[0]
    dst = edge_index[1]
    msg = xw[src] * edge_weight[:, None]
    h_acc = jax.ops.segment_sum(msg, dst, num_segments=N)
    degs = jax.ops.segment_sum(jnp.ones((E,), jnp.float32), dst, num_segments=N)
    norm = jnp.where(degs > 0, 1.0 / degs, 0.0)
    h = h_acc * norm[:, None] + base

    s_list = []
    for i in range(NM):
        rows = motif_idx[i, 0]
        cols = motif_idx[i, 1]
        contrib = h[cols] * motif_val[i][:, None]
        s_list.append(jax.ops.segment_sum(contrib, rows, num_segments=N))
    s = jnp.stack(s_list)

    h_pair = jnp.stack([h[:, :H], h[:, H:]])
    s_all = jnp.stack([s[:, :, :H], s[:, :, H:]], axis=1)

    vc = _build_combined_weights(wa, motif_w)
    cb = motif_b.reshape(1, NM * D)
    mb = jnp.tile(ba, NM).reshape(1, NM * D)
    return _attn(h_pair, s_all, vc, cb, mb)


# speedup vs baseline: 1.2924x; 1.2905x over previous
import functools
import jax
import jax.numpy as jnp
from jax import lax
from jax.experimental import pallas as pl
from jax.experimental.pallas import tpu as pltpu
from jax.experimental.pallas import tpu_sc as plsc

N = 10000
E = 320000
C = 128
D = 32
NM = 13
NNZ = 320000
H = 64
RB = 1000
NRB = N // RB


def _dense_a_kernel(x_ref, w_ref, r_ref, b_ref, xw_ref, base_ref):
    xb = x_ref[...]
    xw_ref[0] = jnp.dot(xb, w_ref[0], preferred_element_type=jnp.float32)
    base_ref[0] = jnp.dot(xb, r_ref[0], preferred_element_type=jnp.float32) + b_ref[0]


def _dense_a(x, W_conv, root, bias):
    w_split = W_conv.reshape(C, 2, H).transpose(1, 0, 2)
    r_split = root.reshape(C, 2, H).transpose(1, 0, 2)
    b_split = bias.reshape(2, 1, H)
    return pl.pallas_call(
        _dense_a_kernel,
        grid=(2, NRB),
        in_specs=[
            pl.BlockSpec((RB, C), lambda c, i: (i, 0)),
            pl.BlockSpec((1, C, H), lambda c, i: (c, 0, 0)),
            pl.BlockSpec((1, C, H), lambda c, i: (c, 0, 0)),
            pl.BlockSpec((1, 1, H), lambda c, i: (c, 0, 0)),
        ],
        out_specs=[
            pl.BlockSpec((1, RB, H), lambda c, i: (c, i, 0)),
            pl.BlockSpec((1, RB, H), lambda c, i: (c, i, 0)),
        ],
        out_shape=[
            jax.ShapeDtypeStruct((2, NP, H), jnp.float32),
            jax.ShapeDtypeStruct((2, NP, H), jnp.float32),
        ],
    )(x, w_split, r_split, b_split)


def _attn_kernel(h_ref, s_ref, vc_ref, cb_ref, mb_ref, o_ref):
    acc = jnp.zeros((RB, 2 * NM * D), jnp.float32)
    for m in range(NM + 1):
        if m == 0:
            rm = jnp.concatenate([h_ref[0], h_ref[1]], axis=1)
        else:
            rm = jnp.concatenate([s_ref[m - 1, 0], s_ref[m - 1, 1]], axis=1)
        acc = acc + jnp.dot(rm, vc_ref[m], preferred_element_type=jnp.float32)
    zc = acc[:, : NM * D] + cb_ref[...]
    zm = acc[:, NM * D :] + mb_ref[...]
    g = (lax.broadcasted_iota(jnp.int32, (NM * D, NM), 0) // D
         == lax.broadcasted_iota(jnp.int32, (NM * D, NM), 1)).astype(jnp.float32)
    logits = jnp.dot(zc * zm, g, preferred_element_type=jnp.float32)
    att = jax.nn.sigmoid(logits)
    att_e = jnp.dot(att, g.T, preferred_element_type=jnp.float32)
    o_ref[...] = att_e * (zm - zc)


def _attn(h_pair, s_all, vc, cb, mb):
    return pl.pallas_call(
        _attn_kernel,
        grid=(NRB,),
        in_specs=[
            pl.BlockSpec((2, RB, H), lambda i: (0, i, 0)),
            pl.BlockSpec((NM, 2, RB, H), lambda i: (0, 0, i, 0)),
            pl.BlockSpec((NM + 1, C, 2 * NM * D), lambda i: (0, 0, 0)),
            pl.BlockSpec((1, NM * D), lambda i: (0, 0)),
            pl.BlockSpec((1, NM * D), lambda i: (0, 0)),
        ],
        out_specs=pl.BlockSpec((RB, NM * D), lambda i: (i, 0)),
        out_shape=jax.ShapeDtypeStruct((N, NM * D), jnp.float32),
    )(h_pair, s_all, vc, cb, mb)


NS = 16
ET = E // NS
NP = 10240
RT = NP // NS
K = 80
NCHUNK = ET // K
NQ = 32


def _sc_body(xw_hbm, base_hbm, src_hbm, dst_hbm, ew_hbm, mrows_hbm, mcols_hbm, mval_hbm,
             h_out, s_out,
             gidx, oidx, vbuf, grows, nbuf, bbuf, hbuf, zbuf, dub, degbuf,
             acc, deg2d, sem):
    c = lax.axis_index("c")
    s = lax.axis_index("s")
    r0 = s * RT

    z16 = jnp.zeros((16,), jnp.float32)
    dconst = jnp.where(jnp.arange(16, dtype=jnp.int32) == 0, 1.0, 0.0).astype(jnp.float32)

    def zrow(j, _):
        for f in range(H // 16):
            zbuf[j, pl.ds(f * 16, 16)] = z16
        degbuf[j, pl.ds(0, 16)] = z16
        return 0
    lax.fori_loop(0, NQ, zrow, 0)
    def drow(j, _):
        dub[j, pl.ds(0, 16)] = dconst
        return 0
    lax.fori_loop(0, K, drow, 0)
    def zacc(q, _):
        pltpu.sync_copy(zbuf, acc.at[pl.ds(r0 + q * NQ, NQ)])
        pltpu.sync_copy(degbuf, deg2d.at[pl.ds(r0 + q * NQ, NQ)])
        return 0
    lax.fori_loop(0, RT // NQ, zacc, 0)
    plsc.subcore_barrier()

    def edge_chunk(g, _):
        e0 = s * ET + g * K
        pltpu.sync_copy(src_hbm.at[pl.ds(e0, K)], gidx)
        pltpu.sync_copy(dst_hbm.at[pl.ds(e0, K)], oidx)
        pltpu.sync_copy(ew_hbm.at[pl.ds(e0, K)], vbuf)
        pltpu.async_copy(xw_hbm.at[c].at[gidx], grows, sem).wait()
        def scale(t, _):
            j0 = t * 16
            vv = vbuf[pl.ds(j0, 16)]
            for jj in range(16):
                v = jnp.full((16,), vv[jj], jnp.float32)
                for f in range(H // 16):
                    sl = pl.ds(f * 16, 16)
                    grows[j0 + jj, sl] = grows[j0 + jj, sl] * v
            return 0
        lax.fori_loop(0, K // 16, scale, 0)
        pltpu.sync_copy(grows, acc.at[oidx], add=True)
        pltpu.sync_copy(dub, deg2d.at[oidx], add=True)
        return 0
    lax.fori_loop(0, NCHUNK, edge_chunk, 0)
    plsc.subcore_barrier()

    def norm_q(q, _):
        r = r0 + q * NQ
        pltpu.sync_copy(acc.at[pl.ds(r, NQ)], nbuf)
        pltpu.sync_copy(zbuf, acc.at[pl.ds(r, NQ)])
        pltpu.sync_copy(base_hbm.at[c, pl.ds(r, NQ)], bbuf)
        pltpu.sync_copy(deg2d.at[pl.ds(r, NQ)], degbuf)
        def nrow(j, _):
            dv = degbuf[j, pl.ds(0, 16)]
            ivv = jnp.where(dv > 0.0, 1.0 / dv, 0.0)
            iv = jnp.full((16,), ivv[0], jnp.float32)
            for f in range(H // 16):
                sl = pl.ds(f * 16, 16)
                hbuf[j, sl] = nbuf[j, sl] * iv + bbuf[j, sl]
            return 0
        lax.fori_loop(0, NQ, nrow, 0)
        pltpu.sync_copy(hbuf, h_out.at[c, pl.ds(r, NQ)])
        return 0
    lax.fori_loop(0, RT // NQ, norm_q, 0)
    plsc.subcore_barrier()

    def motif_body(i, _):
        def chunk(g, __):
            e0 = i * NNZ + s * ET + g * K
            pltpu.sync_copy(mcols_hbm.at[pl.ds(e0, K)], gidx)
            pltpu.sync_copy(mrows_hbm.at[pl.ds(e0, K)], oidx)
            pltpu.sync_copy(mval_hbm.at[pl.ds(e0, K)], vbuf)
            pltpu.async_copy(h_out.at[c].at[gidx], grows, sem).wait()
            def scale(t, ___):
                j0 = t * 16
                vv = vbuf[pl.ds(j0, 16)]
                for jj in range(16):
                    v = jnp.full((16,), vv[jj], jnp.float32)
                    for f in range(H // 16):
                        sl = pl.ds(f * 16, 16)
                        grows[j0 + jj, sl] = grows[j0 + jj, sl] * v
                return 0
            lax.fori_loop(0, K // 16, scale, 0)
            pltpu.sync_copy(grows, acc.at[oidx], add=True)
            return 0
        lax.fori_loop(0, NCHUNK, chunk, 0)
        plsc.subcore_barrier()
        pltpu.sync_copy(acc.at[pl.ds(r0, RT)], s_out.at[i, c, pl.ds(r0, RT)])
        def zacc2(q, __):
            pltpu.sync_copy(zbuf, acc.at[pl.ds(r0 + q * NQ, NQ)])
            return 0
        lax.fori_loop(0, RT // NQ, zacc2, 0)
        plsc.subcore_barrier()
        return 0
    lax.fori_loop(0, NM, motif_body, 0)


def _sc_sparse(xw_pair, base_pair, src, dst, ew, mrows, mcols, mval):
    mesh = plsc.VectorSubcoreMesh(core_axis_name="c", subcore_axis_name="s")
    f = pl.kernel(
        _sc_body,
        out_type=[
            jax.ShapeDtypeStruct((2, NP, H), jnp.float32),
            jax.ShapeDtypeStruct((NM, 2, NP, H), jnp.float32),
        ],
        mesh=mesh,
        compiler_params=pltpu.CompilerParams(use_tc_tiling_on_sc=False),
        scratch_types=[
            pltpu.VMEM((K,), jnp.int32),
            pltpu.VMEM((K,), jnp.int32),
            pltpu.VMEM((K,), jnp.float32),
            pltpu.VMEM((K, H), jnp.float32),
            pltpu.VMEM((NQ, H), jnp.float32),
            pltpu.VMEM((NQ, H), jnp.float32),
            pltpu.VMEM((NQ, H), jnp.float32),
            pltpu.VMEM((NQ, H), jnp.float32),
            pltpu.VMEM((K, 16), jnp.float32),
            pltpu.VMEM((NQ, 16), jnp.float32),
            pltpu.VMEM_SHARED((NP, H), jnp.float32),
            pltpu.VMEM_SHARED((NP, 16), jnp.float32),
            pltpu.SemaphoreType.DMA,
        ],
    )
    return f(xw_pair, base_pair, src, dst, ew, mrows, mcols, mval)


def _build_combined_weights(wa, motif_w):
    blocks = motif_w.reshape(NM, NM, C, D)
    vc = jnp.zeros((NM + 1, C, 2 * NM * D), jnp.float32)
    for i in range(1, NM + 1):
        for j in range(NM + 1):
            if j == i:
                continue
            jj = j if j < i else j - 1
            vc = vc.at[j, :, (i - 1) * D : i * D].set(blocks[i - 1, jj])
        vc = vc.at[i, :, NM * D + (i - 1) * D : NM * D + i * D].set(wa)
    return vc


def kernel(x, edge_weight, motif_val, W_conv, root, bias, wa, ba, motif_w, motif_b, edge_index, motif_idx):
    xw_pair, base_pair = _dense_a(x, W_conv, root, bias)

    h_pair, s_all = _sc_sparse(
        xw_pair, base_pair,
        edge_index[0], edge_index[1], edge_weight,
        motif_idx[:, 0].reshape(-1), motif_idx[:, 1].reshape(-1),
        motif_val.reshape(-1),
    )

    vc = _build_combined_weights(wa, motif_w)
    cb = motif_b.reshape(1, NM * D)
    mb = jnp.tile(ba, NM).reshape(1, NM * D)
    return _attn(h_pair, s_all, vc, cb, mb)


# speedup vs baseline: 4.5773x; 3.5417x over previous
import functools
import jax
import jax.numpy as jnp
from jax import lax
from jax.experimental import pallas as pl
from jax.experimental.pallas import tpu as pltpu
from jax.experimental.pallas import tpu_sc as plsc

N = 10000
E = 320000
C = 128
D = 32
NM = 13
NNZ = 320000
H = 64
RB = 1000
NRB = N // RB


def _dense_a_kernel(x_ref, w_ref, r_ref, b_ref, xw_ref, base_ref):
    xb = x_ref[...]
    xw_ref[0] = jnp.dot(xb, w_ref[0], preferred_element_type=jnp.float32)
    base_ref[0] = jnp.dot(xb, r_ref[0], preferred_element_type=jnp.float32) + b_ref[0]


def _dense_a(x, W_conv, root, bias):
    w_split = W_conv.reshape(C, 2, H).transpose(1, 0, 2)
    r_split = root.reshape(C, 2, H).transpose(1, 0, 2)
    b_split = bias.reshape(2, 1, H)
    return pl.pallas_call(
        _dense_a_kernel,
        grid=(2, NRB),
        in_specs=[
            pl.BlockSpec((RB, C), lambda c, i: (i, 0)),
            pl.BlockSpec((1, C, H), lambda c, i: (c, 0, 0)),
            pl.BlockSpec((1, C, H), lambda c, i: (c, 0, 0)),
            pl.BlockSpec((1, 1, H), lambda c, i: (c, 0, 0)),
        ],
        out_specs=[
            pl.BlockSpec((1, RB, H), lambda c, i: (c, i, 0)),
            pl.BlockSpec((1, RB, H), lambda c, i: (c, i, 0)),
        ],
        out_shape=[
            jax.ShapeDtypeStruct((2, NP, H), jnp.float32),
            jax.ShapeDtypeStruct((2, NP, H), jnp.float32),
        ],
    )(x, w_split, r_split, b_split)


def _attn_kernel(h_ref, s_ref, vc_ref, cb_ref, mb_ref, o_ref):
    acc = jnp.zeros((RB, 2 * NM * D), jnp.float32)
    for m in range(NM + 1):
        if m == 0:
            rm = jnp.concatenate([h_ref[0], h_ref[1]], axis=1)
        else:
            rm = jnp.concatenate([s_ref[m - 1, 0], s_ref[m - 1, 1]], axis=1)
        acc = acc + jnp.dot(rm, vc_ref[m], preferred_element_type=jnp.float32)
    zc = acc[:, : NM * D] + cb_ref[...]
    zm = acc[:, NM * D :] + mb_ref[...]
    g = (lax.broadcasted_iota(jnp.int32, (NM * D, NM), 0) // D
         == lax.broadcasted_iota(jnp.int32, (NM * D, NM), 1)).astype(jnp.float32)
    logits = jnp.dot(zc * zm, g, preferred_element_type=jnp.float32)
    att = jax.nn.sigmoid(logits)
    att_e = jnp.dot(att, g.T, preferred_element_type=jnp.float32)
    o_ref[...] = att_e * (zm - zc)


def _attn(h_pair, s_all, vc, cb, mb):
    return pl.pallas_call(
        _attn_kernel,
        grid=(NRB,),
        in_specs=[
            pl.BlockSpec((2, RB, H), lambda i: (0, i, 0)),
            pl.BlockSpec((NM, 2, RB, H), lambda i: (0, 0, i, 0)),
            pl.BlockSpec((NM + 1, C, 2 * NM * D), lambda i: (0, 0, 0)),
            pl.BlockSpec((1, NM * D), lambda i: (0, 0)),
            pl.BlockSpec((1, NM * D), lambda i: (0, 0)),
        ],
        out_specs=pl.BlockSpec((RB, NM * D), lambda i: (i, 0)),
        out_shape=jax.ShapeDtypeStruct((N, NM * D), jnp.float32),
    )(h_pair, s_all, vc, cb, mb)


NS = 16
ET = E // NS
NP = 10240
RT = NP // NS
K = 80
NCHUNK = ET // K
NQ = 32


def _sc_body(xw_hbm, base_hbm, src_hbm, dst_hbm, ew_hbm, mrows_hbm, mcols_hbm, mval_hbm,
             h_out, s_out,
             gidx2, oidx2, vbuf2, grows2, nbuf, bbuf, hbuf, zbuf, dub, degbuf,
             acc, deg2d, semi, semg, sems):
    c = lax.axis_index("c")
    s = lax.axis_index("s")
    r0 = s * RT

    z16 = jnp.zeros((16,), jnp.float32)
    dconst = jnp.where(jnp.arange(16, dtype=jnp.int32) == 0, 1.0, 0.0).astype(jnp.float32)

    def zrow(j, _):
        for f in range(H // 16):
            zbuf[j, pl.ds(f * 16, 16)] = z16
        degbuf[j, pl.ds(0, 16)] = z16
        return 0
    lax.fori_loop(0, NQ, zrow, 0)
    def drow(j, _):
        dub[j, pl.ds(0, 16)] = dconst
        return 0
    lax.fori_loop(0, K, drow, 0)
    def zacc(q, _):
        pltpu.sync_copy(zbuf, acc.at[pl.ds(r0 + q * NQ, NQ)])
        pltpu.sync_copy(degbuf, deg2d.at[pl.ds(r0 + q * NQ, NQ)])
        return 0
    lax.fori_loop(0, RT // NQ, zacc, 0)
    plsc.subcore_barrier()

    def run_pipeline(e0_of, csrc, rsrc, vsrc, gtab, with_deg):
        def load(g, b):
            e0 = e0_of(g)
            pltpu.async_copy(csrc.at[pl.ds(e0, K)], gidx2.at[b], semi)
            pltpu.async_copy(rsrc.at[pl.ds(e0, K)], oidx2.at[b], semi)
            pltpu.async_copy(vsrc.at[pl.ds(e0, K)], vbuf2.at[b], semi)
        def wait_load(b):
            pltpu.make_async_copy(csrc.at[pl.ds(0, K)], gidx2.at[b], semi).wait()
            pltpu.make_async_copy(rsrc.at[pl.ds(0, K)], oidx2.at[b], semi).wait()
            pltpu.make_async_copy(vsrc.at[pl.ds(0, K)], vbuf2.at[b], semi).wait()
        def gather_start(b):
            pltpu.async_copy(gtab.at[gidx2.at[b]], grows2.at[b], semg.at[b])
        def wait_gather(b):
            pltpu.make_async_copy(gtab.at[pl.ds(0, K)], grows2.at[b], semg.at[b]).wait()
        def scatter_start(b):
            pltpu.async_copy(grows2.at[b], acc.at[oidx2.at[b]], sems.at[b], add=True)
            if with_deg:
                pltpu.async_copy(dub, deg2d.at[oidx2.at[b]], sems.at[b], add=True)
        def wait_scatter(b):
            pltpu.make_async_copy(grows2.at[b], acc.at[pl.ds(0, K)], sems.at[b]).wait()
            if with_deg:
                pltpu.make_async_copy(dub, deg2d.at[pl.ds(0, K)], sems.at[b]).wait()
        def scale(b):
            def blk(t, _):
                j0 = t * 16
                vv = vbuf2[b, pl.ds(j0, 16)]
                for jj in range(16):
                    v = jnp.full((16,), vv[jj], jnp.float32)
                    for f in range(H // 16):
                        sl = pl.ds(f * 16, 16)
                        grows2[b, j0 + jj, sl] = grows2[b, j0 + jj, sl] * v
                return 0
            lax.fori_loop(0, K // 16, blk, 0)

        load(0, 0)
        wait_load(0)
        gather_start(0)
        load(1, 1)
        def body(g, _):
            b = jnp.bitwise_and(g, 1)
            nb = 1 - b
            @pl.when(g + 1 < NCHUNK)
            def _():
                wait_load(nb)
                @pl.when(g >= 1)
                def _():
                    wait_scatter(nb)
                gather_start(nb)
            wait_gather(b)
            scale(b)
            scatter_start(b)
            @pl.when(g + 2 < NCHUNK)
            def _():
                load(g + 2, b)
            return 0
        lax.fori_loop(0, NCHUNK, body, 0)
        wait_scatter(0)
        wait_scatter(1)

    run_pipeline(lambda g: s * ET + g * K, src_hbm, dst_hbm, ew_hbm,
                 xw_hbm.at[c], True)
    plsc.subcore_barrier()

    def norm_q(q, _):
        r = r0 + q * NQ
        pltpu.sync_copy(acc.at[pl.ds(r, NQ)], nbuf)
        pltpu.sync_copy(zbuf, acc.at[pl.ds(r, NQ)])
        pltpu.sync_copy(base_hbm.at[c, pl.ds(r, NQ)], bbuf)
        pltpu.sync_copy(deg2d.at[pl.ds(r, NQ)], degbuf)
        def nrow(j, _):
            dv = degbuf[j, pl.ds(0, 16)]
            ivv = jnp.where(dv > 0.0, 1.0 / dv, 0.0)
            iv = jnp.full((16,), ivv[0], jnp.float32)
            for f in range(H // 16):
                sl = pl.ds(f * 16, 16)
                hbuf[j, sl] = nbuf[j, sl] * iv + bbuf[j, sl]
            return 0
        lax.fori_loop(0, NQ, nrow, 0)
        pltpu.sync_copy(hbuf, h_out.at[c, pl.ds(r, NQ)])
        return 0
    lax.fori_loop(0, RT // NQ, norm_q, 0)
    plsc.subcore_barrier()

    def motif_body(i, _):
        run_pipeline(lambda g: i * NNZ + s * ET + g * K,
                     mcols_hbm, mrows_hbm, mval_hbm, h_out.at[c], False)
        plsc.subcore_barrier()
        pltpu.sync_copy(acc.at[pl.ds(r0, RT)], s_out.at[i, c, pl.ds(r0, RT)])
        def zacc2(q, __):
            pltpu.sync_copy(zbuf, acc.at[pl.ds(r0 + q * NQ, NQ)])
            return 0
        lax.fori_loop(0, RT // NQ, zacc2, 0)
        plsc.subcore_barrier()
        return 0
    lax.fori_loop(0, NM, motif_body, 0)


def _sc_sparse(xw_pair, base_pair, src, dst, ew, mrows, mcols, mval):
    mesh = plsc.VectorSubcoreMesh(core_axis_name="c", subcore_axis_name="s")
    f = pl.kernel(
        _sc_body,
        out_type=[
            jax.ShapeDtypeStruct((2, NP, H), jnp.float32),
            jax.ShapeDtypeStruct((NM, 2, NP, H), jnp.float32),
        ],
        mesh=mesh,
        compiler_params=pltpu.CompilerParams(use_tc_tiling_on_sc=False),
        scratch_types=[
            pltpu.VMEM((2, K), jnp.int32),
            pltpu.VMEM((2, K), jnp.int32),
            pltpu.VMEM((2, K), jnp.float32),
            pltpu.VMEM((2, K, H), jnp.float32),
            pltpu.VMEM((NQ, H), jnp.float32),
            pltpu.VMEM((NQ, H), jnp.float32),
            pltpu.VMEM((NQ, H), jnp.float32),
            pltpu.VMEM((NQ, H), jnp.float32),
            pltpu.VMEM((K, 16), jnp.float32),
            pltpu.VMEM((NQ, 16), jnp.float32),
            pltpu.VMEM_SHARED((NP, H), jnp.float32),
            pltpu.VMEM_SHARED((NP, 16), jnp.float32),
            pltpu.SemaphoreType.DMA,
            pltpu.SemaphoreType.DMA((2,)),
            pltpu.SemaphoreType.DMA((2,)),
        ],
    )
    return f(xw_pair, base_pair, src, dst, ew, mrows, mcols, mval)


def _build_combined_weights(wa, motif_w):
    blocks = motif_w.reshape(NM, NM, C, D)
    vc = jnp.zeros((NM + 1, C, 2 * NM * D), jnp.float32)
    for i in range(1, NM + 1):
        for j in range(NM + 1):
            if j == i:
                continue
            jj = j if j < i else j - 1
            vc = vc.at[j, :, (i - 1) * D : i * D].set(blocks[i - 1, jj])
        vc = vc.at[i, :, NM * D + (i - 1) * D : NM * D + i * D].set(wa)
    return vc


def kernel(x, edge_weight, motif_val, W_conv, root, bias, wa, ba, motif_w, motif_b, edge_index, motif_idx):
    xw_pair, base_pair = _dense_a(x, W_conv, root, bias)

    h_pair, s_all = _sc_sparse(
        xw_pair, base_pair,
        edge_index[0], edge_index[1], edge_weight,
        motif_idx[:, 0].reshape(-1), motif_idx[:, 1].reshape(-1),
        motif_val.reshape(-1),
    )

    vc = _build_combined_weights(wa, motif_w)
    cb = motif_b.reshape(1, NM * D)
    mb = jnp.tile(ba, NM).reshape(1, NM * D)
    return _attn(h_pair, s_all, vc, cb, mb)


# speedup vs baseline: 5.1240x; 1.1194x over previous
import functools
import jax
import jax.numpy as jnp
from jax import lax
from jax.experimental import pallas as pl
from jax.experimental.pallas import tpu as pltpu
from jax.experimental.pallas import tpu_sc as plsc

N = 10000
E = 320000
C = 128
D = 32
NM = 13
NNZ = 320000
H = 64
RB = 1000
NRB = N // RB


def _dense_a_kernel(x_ref, w_ref, r_ref, b_ref, xw_ref, base_ref):
    xb = x_ref[...]
    xw_ref[0] = jnp.dot(xb, w_ref[0], preferred_element_type=jnp.float32)
    base_ref[0] = jnp.dot(xb, r_ref[0], preferred_element_type=jnp.float32) + b_ref[0]


def _dense_a(x, W_conv, root, bias):
    w_split = W_conv.reshape(C, 2, H).transpose(1, 0, 2)
    r_split = root.reshape(C, 2, H).transpose(1, 0, 2)
    b_split = bias.reshape(2, 1, H)
    return pl.pallas_call(
        _dense_a_kernel,
        grid=(2, NRB),
        in_specs=[
            pl.BlockSpec((RB, C), lambda c, i: (i, 0)),
            pl.BlockSpec((1, C, H), lambda c, i: (c, 0, 0)),
            pl.BlockSpec((1, C, H), lambda c, i: (c, 0, 0)),
            pl.BlockSpec((1, 1, H), lambda c, i: (c, 0, 0)),
        ],
        out_specs=[
            pl.BlockSpec((1, RB, H), lambda c, i: (c, i, 0)),
            pl.BlockSpec((1, RB, H), lambda c, i: (c, i, 0)),
        ],
        out_shape=[
            jax.ShapeDtypeStruct((2, NP, H), jnp.float32),
            jax.ShapeDtypeStruct((2, NP, H), jnp.float32),
        ],
    )(x, w_split, r_split, b_split)


def _attn_kernel(h_ref, s_ref, vc_ref, cb_ref, mb_ref, o_ref):
    acc = jnp.zeros((RB, 2 * NM * D), jnp.float32)
    for m in range(NM + 1):
        if m == 0:
            rm = jnp.concatenate([h_ref[0], h_ref[1]], axis=1)
        else:
            rm = jnp.concatenate([s_ref[m - 1, 0], s_ref[m - 1, 1]], axis=1)
        acc = acc + jnp.dot(rm, vc_ref[m], preferred_element_type=jnp.float32)
    zc = acc[:, : NM * D] + cb_ref[...]
    zm = acc[:, NM * D :] + mb_ref[...]
    g = (lax.broadcasted_iota(jnp.int32, (NM * D, NM), 0) // D
         == lax.broadcasted_iota(jnp.int32, (NM * D, NM), 1)).astype(jnp.float32)
    logits = jnp.dot(zc * zm, g, preferred_element_type=jnp.float32)
    att = jax.nn.sigmoid(logits)
    att_e = jnp.dot(att, g.T, preferred_element_type=jnp.float32)
    o_ref[...] = att_e * (zm - zc)


def _attn(h_pair, s_all, vc, cb, mb):
    return pl.pallas_call(
        _attn_kernel,
        grid=(NRB,),
        in_specs=[
            pl.BlockSpec((2, RB, H), lambda i: (0, i, 0)),
            pl.BlockSpec((NM, 2, RB, H), lambda i: (0, 0, i, 0)),
            pl.BlockSpec((NM + 1, C, 2 * NM * D), lambda i: (0, 0, 0)),
            pl.BlockSpec((1, NM * D), lambda i: (0, 0)),
            pl.BlockSpec((1, NM * D), lambda i: (0, 0)),
        ],
        out_specs=pl.BlockSpec((RB, NM * D), lambda i: (i, 0)),
        out_shape=jax.ShapeDtypeStruct((N, NM * D), jnp.float32),
    )(h_pair, s_all, vc, cb, mb)


NS = 16
ET = E // NS
NP = 10240
RT = NP // NS
K = 80
NCHUNK = ET // K
NQ = 32


def _sc_body(xw_hbm, base_hbm, src_hbm, dst_hbm, ew_hbm, mrows_hbm, mcols_hbm, mval_hbm,
             h_out, s_out,
             gidx2, oidx2, vbuf2, grows2, nbuf, bbuf, hbuf, zbuf, dub, degbuf,
             acc, deg2d, semi, semg, sems):
    c = lax.axis_index("c")
    s = lax.axis_index("s")
    r0 = s * RT

    z16 = jnp.zeros((16,), jnp.float32)
    dconst = jnp.where(jnp.arange(16, dtype=jnp.int32) == 0, 1.0, 0.0).astype(jnp.float32)

    def zrow(j, _):
        for f in range(H // 16):
            zbuf[j, pl.ds(f * 16, 16)] = z16
        degbuf[j, pl.ds(0, 16)] = z16
        return 0
    lax.fori_loop(0, NQ, zrow, 0)
    def drow(j, _):
        dub[j, pl.ds(0, 16)] = dconst
        return 0
    lax.fori_loop(0, K, drow, 0)
    def zacc(q, _):
        pltpu.sync_copy(zbuf, acc.at[pl.ds(r0 + q * NQ, NQ)])
        pltpu.sync_copy(degbuf, deg2d.at[pl.ds(r0 + q * NQ, NQ)])
        return 0
    lax.fori_loop(0, RT // NQ, zacc, 0)
    plsc.subcore_barrier()

    def run_pipeline(e0_of, csrc, rsrc, vsrc, gtab, with_deg):
        def load(g, b):
            e0 = e0_of(g)
            pltpu.async_copy(csrc.at[pl.ds(e0, K)], gidx2.at[b], semi)
            pltpu.async_copy(rsrc.at[pl.ds(e0, K)], oidx2.at[b], semi)
            pltpu.async_copy(vsrc.at[pl.ds(e0, K)], vbuf2.at[b], semi)
        def wait_load(b):
            pltpu.make_async_copy(csrc.at[pl.ds(0, K)], gidx2.at[b], semi).wait()
            pltpu.make_async_copy(rsrc.at[pl.ds(0, K)], oidx2.at[b], semi).wait()
            pltpu.make_async_copy(vsrc.at[pl.ds(0, K)], vbuf2.at[b], semi).wait()
        def gather_start(b):
            pltpu.async_copy(gtab.at[gidx2.at[b]], grows2.at[b], semg.at[b])
        def wait_gather(b):
            pltpu.make_async_copy(gtab.at[pl.ds(0, K)], grows2.at[b], semg.at[b]).wait()
        def scatter_start(b):
            pltpu.async_copy(grows2.at[b], acc.at[oidx2.at[b]], sems.at[b], add=True)
            if with_deg:
                pltpu.async_copy(dub, deg2d.at[oidx2.at[b]], sems.at[b], add=True)
        def wait_scatter(b):
            pltpu.make_async_copy(grows2.at[b], acc.at[pl.ds(0, K)], sems.at[b]).wait()
            if with_deg:
                pltpu.make_async_copy(dub, deg2d.at[pl.ds(0, K)], sems.at[b]).wait()
        def scale(b):
            @plsc.parallel_loop(0, K // 16, unroll=K // 16)
            def blk(t):
                j0 = t * 16
                vv = vbuf2[b, pl.ds(j0, 16)]
                for jj in range(16):
                    v = jnp.full((16,), vv[jj], jnp.float32)
                    for f in range(H // 16):
                        sl = pl.ds(f * 16, 16)
                        grows2[b, j0 + jj, sl] = grows2[b, j0 + jj, sl] * v

        load(0, 0)
        wait_load(0)
        gather_start(0)
        load(1, 1)
        def body(g, _):
            b = jnp.bitwise_and(g, 1)
            nb = 1 - b
            @pl.when(g + 1 < NCHUNK)
            def _():
                wait_load(nb)
                @pl.when(g >= 1)
                def _():
                    wait_scatter(nb)
                gather_start(nb)
            wait_gather(b)
            scale(b)
            scatter_start(b)
            @pl.when(g + 2 < NCHUNK)
            def _():
                load(g + 2, b)
            return 0
        lax.fori_loop(0, NCHUNK, body, 0)
        wait_scatter(0)
        wait_scatter(1)

    run_pipeline(lambda g: s * ET + g * K, src_hbm, dst_hbm, ew_hbm,
                 xw_hbm.at[c], True)
    plsc.subcore_barrier()

    def norm_q(q, _):
        r = r0 + q * NQ
        pltpu.sync_copy(acc.at[pl.ds(r, NQ)], nbuf)
        pltpu.sync_copy(zbuf, acc.at[pl.ds(r, NQ)])
        pltpu.sync_copy(base_hbm.at[c, pl.ds(r, NQ)], bbuf)
        pltpu.sync_copy(deg2d.at[pl.ds(r, NQ)], degbuf)
        def nrow(j, _):
            dv = degbuf[j, pl.ds(0, 16)]
            ivv = jnp.where(dv > 0.0, 1.0 / dv, 0.0)
            iv = jnp.full((16,), ivv[0], jnp.float32)
            for f in range(H // 16):
                sl = pl.ds(f * 16, 16)
                hbuf[j, sl] = nbuf[j, sl] * iv + bbuf[j, sl]
            return 0
        lax.fori_loop(0, NQ, nrow, 0)
        pltpu.sync_copy(hbuf, h_out.at[c, pl.ds(r, NQ)])
        return 0
    lax.fori_loop(0, RT // NQ, norm_q, 0)
    plsc.subcore_barrier()

    def motif_body(i, _):
        run_pipeline(lambda g: i * NNZ + s * ET + g * K,
                     mcols_hbm, mrows_hbm, mval_hbm, h_out.at[c], False)
        plsc.subcore_barrier()
        pltpu.sync_copy(acc.at[pl.ds(r0, RT)], s_out.at[i, c, pl.ds(r0, RT)])
        def zacc2(q, __):
            pltpu.sync_copy(zbuf, acc.at[pl.ds(r0 + q * NQ, NQ)])
            return 0
        lax.fori_loop(0, RT // NQ, zacc2, 0)
        plsc.subcore_barrier()
        return 0
    lax.fori_loop(0, NM, motif_body, 0)


def _sc_sparse(xw_pair, base_pair, src, dst, ew, mrows, mcols, mval):
    mesh = plsc.VectorSubcoreMesh(core_axis_name="c", subcore_axis_name="s")
    f = pl.kernel(
        _sc_body,
        out_type=[
            jax.ShapeDtypeStruct((2, NP, H), jnp.float32),
            jax.ShapeDtypeStruct((NM, 2, NP, H), jnp.float32),
        ],
        mesh=mesh,
        compiler_params=pltpu.CompilerParams(use_tc_tiling_on_sc=False),
        scratch_types=[
            pltpu.VMEM((2, K), jnp.int32),
            pltpu.VMEM((2, K), jnp.int32),
            pltpu.VMEM((2, K), jnp.float32),
            pltpu.VMEM((2, K, H), jnp.float32),
            pltpu.VMEM((NQ, H), jnp.float32),
            pltpu.VMEM((NQ, H), jnp.float32),
            pltpu.VMEM((NQ, H), jnp.float32),
            pltpu.VMEM((NQ, H), jnp.float32),
            pltpu.VMEM((K, 16), jnp.float32),
            pltpu.VMEM((NQ, 16), jnp.float32),
            pltpu.VMEM_SHARED((NP, H), jnp.float32),
            pltpu.VMEM_SHARED((NP, 16), jnp.float32),
            pltpu.SemaphoreType.DMA,
            pltpu.SemaphoreType.DMA((2,)),
            pltpu.SemaphoreType.DMA((2,)),
        ],
    )
    return f(xw_pair, base_pair, src, dst, ew, mrows, mcols, mval)


def _build_combined_weights(wa, motif_w):
    blocks = motif_w.reshape(NM, NM, C, D)
    vc = jnp.zeros((NM + 1, C, 2 * NM * D), jnp.float32)
    for i in range(1, NM + 1):
        for j in range(NM + 1):
            if j == i:
                continue
            jj = j if j < i else j - 1
            vc = vc.at[j, :, (i - 1) * D : i * D].set(blocks[i - 1, jj])
        vc = vc.at[i, :, NM * D + (i - 1) * D : NM * D + i * D].set(wa)
    return vc


def kernel(x, edge_weight, motif_val, W_conv, root, bias, wa, ba, motif_w, motif_b, edge_index, motif_idx):
    xw_pair, base_pair = _dense_a(x, W_conv, root, bias)

    h_pair, s_all = _sc_sparse(
        xw_pair, base_pair,
        edge_index[0], edge_index[1], edge_weight,
        motif_idx[:, 0].reshape(-1), motif_idx[:, 1].reshape(-1),
        motif_val.reshape(-1),
    )

    vc = _build_combined_weights(wa, motif_w)
    cb = motif_b.reshape(1, NM * D)
    mb = jnp.tile(ba, NM).reshape(1, NM * D)
    return _attn(h_pair, s_all, vc, cb, mb)
